# baseline (speedup 1.0000x reference)
import functools

import jax
import jax.numpy as jnp
from jax import lax
from jax.experimental import pallas as pl
from jax.experimental.pallas import tpu as pltpu
from jax.experimental.pallas import tpu_sc as plsc

N_NODES = 10000
N_EDGES = 320000
D = 128
NUM_RBF = 16
RBF_MIN = 0.0
RBF_MAX = 12.0
GAMMA = 1.0 / ((RBF_MAX - RBF_MIN) / NUM_RBF) ** 2
MU_STEP = (RBF_MAX - RBF_MIN) / (NUM_RBF - 1)

NC = 2
NS = 16
NW = NC * NS
CH = 80
E_PER_W = N_EDGES // NW
ITERS = E_PER_W // CH


def _node_proj_body(x_ref, wt_ref, y_ref):
    y_ref[...] = 0.5 * jnp.dot(
        x_ref[...], wt_ref[...], preferred_element_type=jnp.float32
    )


def _node_proj(x, w_edge_t):
    return pl.pallas_call(
        _node_proj_body,
        out_shape=jax.ShapeDtypeStruct((N_NODES, D), jnp.float32),
    )(x, w_edge_t)


_sc_mesh = plsc.VectorSubcoreMesh(core_axis_name="c", subcore_axis_name="s")


@functools.partial(
    pl.kernel,
    mesh=_sc_mesh,
    out_type=jax.ShapeDtypeStruct((N_EDGES // 2, D), jnp.float32),
    scratch_types=[
        pltpu.VMEM_SHARED((N_NODES, D), jnp.float32),
        pltpu.VMEM((CH,), jnp.int32),
        pltpu.VMEM((CH,), jnp.int32),
        pltpu.VMEM((CH,), jnp.int32),
        pltpu.VMEM((CH,), jnp.int32),
        pltpu.VMEM((CH, D), jnp.float32),
        pltpu.VMEM((CH, D), jnp.float32),
        pltpu.VMEM((CH, D), jnp.float32),
        pltpu.VMEM((CH, D), jnp.float32),
        pltpu.SemaphoreType.DMA,
        pltpu.SemaphoreType.DMA,
        pltpu.SemaphoreType.DMA,
        pltpu.SemaphoreType.DMA,
        pltpu.SemaphoreType.DMA,
        pltpu.SemaphoreType.DMA,
        pltpu.SemaphoreType.DMA,
        pltpu.SemaphoreType.DMA,
    ],
)
def _sc_gather_sum(y_hbm, src_hbm, dst_hbm, g_hbm,
                   y_sh, ia0, ib0, ia1, ib1, ra0, rb0, ra1, rb1,
                   ga0, gb0, ga1, gb1, ss0, ss1, is0, is1):
    sid = lax.axis_index("s")
    wid = sid * NC + lax.axis_index("c")
    ebase = wid * E_PER_W
    sets = (
        (ia0, ib0, ra0, rb0, ga0, gb0, ss0, is0),
        (ia1, ib1, ra1, rb1, ga1, gb1, ss1, is1),
    )

    slab = 624
    pltpu.sync_copy(
        y_hbm.at[pl.ds(sid * slab, slab)], y_sh.at[pl.ds(sid * slab, slab)]
    )

    @pl.when(sid == NS - 1)
    def _():
        tail = NS * slab
        pltpu.sync_copy(
            y_hbm.at[pl.ds(tail, N_NODES - tail)],
            y_sh.at[pl.ds(tail, N_NODES - tail)],
        )

    plsc.subcore_barrier()

    def issue_idx(j, s):
        ia, ib = sets[s][0], sets[s][1]
        isem = sets[s][7]
        base = ebase + j * CH
        pltpu.async_copy(src_hbm.at[pl.ds(base, CH)], ia, isem)
        pltpu.async_copy(dst_hbm.at[pl.ds(base, CH)], ib, isem)

    def wait_idx(s):
        ia, ib = sets[s][0], sets[s][1]
        isem = sets[s][7]
        pltpu.make_async_copy(src_hbm.at[pl.ds(0, CH)], ia, isem).wait()
        pltpu.make_async_copy(dst_hbm.at[pl.ds(0, CH)], ib, isem).wait()

    def issue_gathers(s):
        ia, ib, ra, rb, ga, gb = sets[s][:6]
        pltpu.async_copy(y_sh.at[ia], ra, ga)
        pltpu.async_copy(y_sh.at[ib], rb, gb)

    def wait_gathers(s):
        ia, ib, ra, rb, ga, gb = sets[s][:6]
        pltpu.make_async_copy(y_sh.at[ia], ra, ga).wait()
        pltpu.make_async_copy(y_sh.at[ib], rb, gb).wait()

    def wait_store(s):
        ra, ss = sets[s][2], sets[s][6]
        pltpu.make_async_copy(
            ra.at[pl.ds(0, CH // 2)], g_hbm.at[pl.ds(0, CH // 2)], ss
        ).wait()

    def add_and_store(j, s):
        ra, rb = sets[s][2], sets[s][3]
        ss = sets[s][6]
        mask_hi = jnp.int32(-65536)

        def pack32(r, k):
            lo = pl.ds(32 * k, 16)
            hi = pl.ds(32 * k + 16, 16)
            a = lax.bitcast_convert_type(ra[r, lo] + rb[r, lo], jnp.int32)
            b = lax.bitcast_convert_type(ra[r, hi] + rb[r, hi], jnp.int32)
            a = a + 0x7FFF + ((a >> 16) & 1)
            b = b + 0x7FFF + ((b >> 16) & 1)
            w = ((a >> 16) & 0xFFFF) | (b & mask_hi)
            return lax.bitcast_convert_type(w, jnp.float32)

        def row_body(m, rcarry):
            for half in range(2):
                r = 2 * m + half
                for k in range(D // 32):
                    ra[m, pl.ds(64 * half + 16 * k, 16)] = pack32(r, k)
            return rcarry

        lax.fori_loop(0, CH // 2, row_body, 0)
        half_base = wid * (E_PER_W // 2) + j * (CH // 2)
        pltpu.async_copy(
            ra.at[pl.ds(0, CH // 2)], g_hbm.at[pl.ds(half_base, CH // 2)], ss
        )

    def sub_iter(j, p):
        q = 1 - p
        wait_gathers(p)

        @pl.when(j <= ITERS - 3)
        def _():
            issue_idx(j + 2, p)

        @pl.when(j <= ITERS - 2)
        def _():
            @pl.when(j >= 1)
            def _():
                wait_store(q)

            wait_idx(q)
            issue_gathers(q)

        add_and_store(j, p)

    issue_idx(0, 0)
    issue_idx(1, 1)
    wait_idx(0)
    issue_gathers(0)

    def pair_body(t, carry):
        sub_iter(2 * t, 0)
        sub_iter(2 * t + 1, 1)
        return carry

    lax.fori_loop(0, (ITERS - 1) // 2, pair_body, 0)
    wait_gathers(0)
    add_and_store(ITERS - 1, 0)

    wait_store(0)
    wait_store(1)


_EB = 6400


def _edge_final_body(g_ref, a_ref, wr_ref, b_ref, o_ref):
    d = a_ref[...]
    mu = (
        lax.broadcasted_iota(jnp.int32, (_EB, NUM_RBF), 1).astype(jnp.float32)
        * MU_STEP
        + RBF_MIN
    )
    diff = d - mu
    rbf = jnp.exp(-GAMMA * diff * diff)
    o_ref[...] = (
        g_ref[...].astype(jnp.float32)
        + jnp.dot(rbf, wr_ref[...], preferred_element_type=jnp.float32)
        + b_ref[...][None, :]
    )


def _edge_final(g, edge_attr_col, w_rbf_t, b_edge):
    n_blocks = N_EDGES // _EB
    return pl.pallas_call(
        _edge_final_body,
        grid=(n_blocks,),
        in_specs=[
            pl.BlockSpec((_EB, D), lambda i: (i, 0)),

            pl.BlockSpec((_EB, 1), lambda i: (i, 0)),
            pl.BlockSpec((NUM_RBF, D), lambda i: (0, 0)),
            pl.BlockSpec((D,), lambda i: (0,)),
        ],
        out_specs=pl.BlockSpec((_EB, D), lambda i: (i, 0)),
        out_shape=jax.ShapeDtypeStruct((N_EDGES, D), jnp.float32),
    )(g, edge_attr_col, w_rbf_t, b_edge)


_PERM = []
for _k in range(D // 32):
    _PERM += [32 * _k + 2 * _i for _i in range(16)]
    _PERM += [32 * _k + 2 * _i + 1 for _i in range(16)]


def kernel(x, edge_index, edge_attr, W_rbf, W_edge, b_edge):
    src = edge_index[0]
    dst = edge_index[1]
    w_et = W_edge.T[:, jnp.asarray(_PERM, dtype=jnp.int32)]
    y = _node_proj(x, w_et)
    g_packed = _sc_gather_sum(y, src, dst)
    g = lax.bitcast_convert_type(g_packed, jnp.bfloat16).reshape(N_EDGES, D)
    return _edge_final(g, edge_attr[:, None], W_rbf.T, b_edge)


# speedup vs baseline: 52.6904x; 52.6904x over previous
import functools

import jax
import jax.numpy as jnp
from jax import lax
from jax.experimental import pallas as pl
from jax.experimental.pallas import tpu as pltpu
from jax.experimental.pallas import tpu_sc as plsc

N_NODES = 10000
N_EDGES = 320000
D = 128
NUM_RBF = 16
RBF_MIN = 0.0
RBF_MAX = 12.0
GAMMA = 1.0 / ((RBF_MAX - RBF_MIN) / NUM_RBF) ** 2
MU_STEP = (RBF_MAX - RBF_MIN) / (NUM_RBF - 1)

NC = 2
NS = 16
NW = NC * NS
CH = 80
E_PER_W = N_EDGES // NW
ITERS = E_PER_W // CH


def _node_proj_body(x_ref, wt_ref, y_ref):
    y_ref[...] = 0.5 * jnp.dot(
        x_ref[...], wt_ref[...], preferred_element_type=jnp.float32
    )


def _node_proj(x, w_edge_t):
    return pl.pallas_call(
        _node_proj_body,
        out_shape=jax.ShapeDtypeStruct((N_NODES, D), jnp.float32),
    )(x, w_edge_t)


_sc_mesh = plsc.VectorSubcoreMesh(core_axis_name="c", subcore_axis_name="s")


@functools.partial(
    pl.kernel,
    mesh=_sc_mesh,
    out_type=jax.ShapeDtypeStruct((N_EDGES, D), jnp.float32),
    scratch_types=[
        pltpu.VMEM_SHARED((N_NODES, D), jnp.float32),
        pltpu.VMEM((CH,), jnp.int32),
        pltpu.VMEM((CH,), jnp.int32),
        pltpu.VMEM((CH,), jnp.int32),
        pltpu.VMEM((CH,), jnp.int32),
        pltpu.VMEM((CH, D), jnp.float32),
        pltpu.VMEM((CH, D), jnp.float32),
        pltpu.VMEM((CH, D), jnp.float32),
        pltpu.VMEM((CH, D), jnp.float32),
        pltpu.SemaphoreType.DMA,
        pltpu.SemaphoreType.DMA,
        pltpu.SemaphoreType.DMA,
        pltpu.SemaphoreType.DMA,
        pltpu.SemaphoreType.DMA,
        pltpu.SemaphoreType.DMA,
        pltpu.SemaphoreType.DMA,
        pltpu.SemaphoreType.DMA,
    ],
)
def _sc_gather_sum(y_hbm, src_hbm, dst_hbm, g_hbm,
                   y_sh, ia0, ib0, ia1, ib1, ra0, rb0, ra1, rb1,
                   ga0, gb0, ga1, gb1, ss0, ss1, is0, is1):
    sid = lax.axis_index("s")
    wid = sid * NC + lax.axis_index("c")
    ebase = wid * E_PER_W
    sets = (
        (ia0, ib0, ra0, rb0, ga0, gb0, ss0, is0),
        (ia1, ib1, ra1, rb1, ga1, gb1, ss1, is1),
    )

    slab = 624
    pltpu.sync_copy(
        y_hbm.at[pl.ds(sid * slab, slab)], y_sh.at[pl.ds(sid * slab, slab)]
    )

    @pl.when(sid == NS - 1)
    def _():
        tail = NS * slab
        pltpu.sync_copy(
            y_hbm.at[pl.ds(tail, N_NODES - tail)],
            y_sh.at[pl.ds(tail, N_NODES - tail)],
        )

    plsc.subcore_barrier()

    def issue_idx(j, s):
        ia, ib = sets[s][0], sets[s][1]
        isem = sets[s][7]
        base = ebase + j * CH
        pltpu.async_copy(src_hbm.at[pl.ds(base, CH)], ia, isem)
        pltpu.async_copy(dst_hbm.at[pl.ds(base, CH)], ib, isem)

    def wait_idx(s):
        ia, ib = sets[s][0], sets[s][1]
        isem = sets[s][7]
        pltpu.make_async_copy(src_hbm.at[pl.ds(0, CH)], ia, isem).wait()
        pltpu.make_async_copy(dst_hbm.at[pl.ds(0, CH)], ib, isem).wait()

    def issue_gathers(s):
        ia, ib, ra, rb, ga, gb = sets[s][:6]
        pltpu.async_copy(y_sh.at[ia], ra, ga)
        pltpu.async_copy(y_sh.at[ib], rb, gb)

    def wait_gathers(s):
        ia, ib, ra, rb, ga, gb = sets[s][:6]
        pltpu.make_async_copy(y_sh.at[ia], ra, ga).wait()
        pltpu.make_async_copy(y_sh.at[ib], rb, gb).wait()

    def wait_store(s):
        ra, ss = sets[s][2], sets[s][6]
        pltpu.make_async_copy(ra, g_hbm.at[pl.ds(0, CH)], ss).wait()

    def add_and_store(j, s):
        ra, rb = sets[s][2], sets[s][3]
        ss = sets[s][6]

        def row_body(r, rcarry):
            for jj in range(D // 16):
                sl = pl.ds(jj * 16, 16)
                ra[r, sl] = ra[r, sl] + rb[r, sl]
            return rcarry

        lax.fori_loop(0, CH, row_body, 0)
        pltpu.async_copy(ra, g_hbm.at[pl.ds(ebase + j * CH, CH)], ss)

    def sub_iter(j, p):
        q = 1 - p
        wait_gathers(p)

        @pl.when(j <= ITERS - 3)
        def _():
            issue_idx(j + 2, p)

        @pl.when(j <= ITERS - 2)
        def _():
            @pl.when(j >= 1)
            def _():
                wait_store(q)

            wait_idx(q)
            issue_gathers(q)

        add_and_store(j, p)

    issue_idx(0, 0)
    issue_idx(1, 1)
    wait_idx(0)
    issue_gathers(0)

    def pair_body(t, carry):
        sub_iter(2 * t, 0)
        sub_iter(2 * t + 1, 1)
        return carry

    lax.fori_loop(0, (ITERS - 1) // 2, pair_body, 0)
    wait_gathers(0)
    add_and_store(ITERS - 1, 0)

    wait_store(0)
    wait_store(1)


_EB = 6400


def _edge_final_body(g_ref, a_ref, wr_ref, b_ref, o_ref):
    d = a_ref[...]
    mu = (
        lax.broadcasted_iota(jnp.int32, (_EB, NUM_RBF), 1).astype(jnp.float32)
        * MU_STEP
        + RBF_MIN
    )
    diff = d - mu
    rbf = jnp.exp(-GAMMA * diff * diff)
    o_ref[...] = (
        g_ref[...]
        + jnp.dot(rbf, wr_ref[...], preferred_element_type=jnp.float32)
        + b_ref[...][None, :]
    )


def _edge_final(g, edge_attr_col, w_rbf_t, b_edge):
    n_blocks = N_EDGES // _EB
    return pl.pallas_call(
        _edge_final_body,
        grid=(n_blocks,),
        in_specs=[
            pl.BlockSpec((_EB, D), lambda i: (i, 0)),
            pl.BlockSpec((_EB, 1), lambda i: (i, 0)),
            pl.BlockSpec((NUM_RBF, D), lambda i: (0, 0)),
            pl.BlockSpec((D,), lambda i: (0,)),
        ],
        out_specs=pl.BlockSpec((_EB, D), lambda i: (i, 0)),
        out_shape=jax.ShapeDtypeStruct((N_EDGES, D), jnp.float32),
    )(g, edge_attr_col, w_rbf_t, b_edge)


def kernel(x, edge_index, edge_attr, W_rbf, W_edge, b_edge):
    src = edge_index[0]
    dst = edge_index[1]
    y = _node_proj(x, W_edge.T)
    g = _sc_gather_sum(y, src, dst)
    return _edge_final(g, edge_attr[:, None], W_rbf.T, b_edge)
